# baseline (speedup 1.0000x reference)
import jax
import jax.numpy as jnp
from jax.experimental import pallas as pl
from jax.experimental.pallas import tpu as pltpu

DIM = 128
BALL = 128
N = 2048
NB = N // BALL
HID = DIM * 4
EPS = float(jnp.finfo(jnp.float32).eps)
SCALE = DIM ** -0.5
NEG = -1e30

QT = 512
GRID = N // QT

_DN = (((1,), (1,)), ((), ()))


def _body(x_ref, pos_ref, n1_ref, n2_ref, w1w_ref, w1b_ref, w2w_ref,
          w2b_ref, w3w_ref, w3b_ref, o_ref, x1_ref, sel_ref):
    i = pl.program_id(0)

    @pl.when(i == 0)
    def _prep():
        x = x_ref[...]
        p = pos_ref[...].reshape(NB, BALL)
        rel = (p - jnp.mean(p, axis=1, keepdims=True)).reshape(N, 1)
        v = jnp.mean(x * x, axis=-1, keepdims=True)
        x1 = x * jax.lax.rsqrt(v + EPS) * n1_ref[...] + rel
        x1_ref[...] = x1
        bm = jnp.mean(x1.reshape(NB, BALL, DIM), axis=1)
        sim = jax.lax.dot_general(x1, bm, _DN,
                                  preferred_element_type=jnp.float32)
        idx = jax.lax.broadcasted_iota(jnp.int32, (N, NB), 1)
        m1 = jnp.max(sim, axis=-1, keepdims=True)
        i1 = jnp.min(jnp.where(sim == m1, idx, NB), axis=-1, keepdims=True)
        sel1 = idx == i1
        sim2 = jnp.where(sel1, NEG, sim)
        m2 = jnp.max(sim2, axis=-1, keepdims=True)
        i2 = jnp.min(jnp.where(sim2 == m2, idx, NB), axis=-1, keepdims=True)
        sel_ref[...] = jnp.where(sel1 | (idx == i2), 0.0, NEG)

    x1 = x1_ref[...]
    x1b = x1.astype(jnp.bfloat16)
    xq = x1_ref[pl.ds(i * QT, QT), :]

    rr = jax.lax.broadcasted_iota(jnp.int32, (NB, N), 0)
    cc = jax.lax.broadcasted_iota(jnp.int32, (NB, N), 1) // BALL
    colmap = (rr == cc).astype(jnp.bfloat16)
    bias = jax.lax.dot_general(
        sel_ref[pl.ds(i * QT, QT), :].astype(jnp.bfloat16), colmap,
        (((1,), (0,)), ((), ())),
        preferred_element_type=jnp.float32)

    s = jax.lax.dot_general(xq * SCALE, x1, _DN,
                            preferred_element_type=jnp.float32)
    s = s + bias
    m = jnp.max(s, axis=-1, keepdims=True)
    p = jnp.exp(s - m)
    attn = jax.lax.dot_general(p.astype(jnp.bfloat16), x1b,
                               (((1,), (0,)), ((), ())),
                               preferred_element_type=jnp.float32)
    attn = attn / jnp.sum(p, axis=-1, keepdims=True)

    x2 = x_ref[pl.ds(i * QT, QT), :] + attn
    v2 = jnp.mean(x2 * x2, axis=-1, keepdims=True)
    xn = (x2 * jax.lax.rsqrt(v2 + EPS) * n2_ref[...]).astype(jnp.bfloat16)
    a = jax.lax.dot_general(xn, w1w_ref[...], _DN,
                            preferred_element_type=jnp.float32) + w1b_ref[...]
    b = jax.lax.dot_general(xn, w2w_ref[...], _DN,
                            preferred_element_type=jnp.float32) + w2b_ref[...]
    h = (b * (a * jax.nn.sigmoid(a))).astype(jnp.bfloat16)
    o_ref[...] = x2 + jax.lax.dot_general(
        h, w3w_ref[...], _DN, preferred_element_type=jnp.float32) + w3b_ref[...]


def kernel(x, pos, batch_idx, norm1_w, norm2_w, w1_w, w1_b, w2_w, w2_b,
           w3_w, w3_b):
    del batch_idx
    full = lambda shape: pl.BlockSpec(shape, lambda i: (0, 0))
    out = pl.pallas_call(
        _body,
        grid=(GRID,),
        in_specs=[
            full((N, DIM)),
            full((N, 1)),
            full((1, DIM)),
            full((1, DIM)),
            full((HID, DIM)),
            full((1, HID)),
            full((HID, DIM)),
            full((1, HID)),
            full((DIM, HID)),
            full((1, DIM)),
        ],
        out_specs=pl.BlockSpec((QT, DIM), lambda i: (i, 0)),
        out_shape=jax.ShapeDtypeStruct((N, DIM), jnp.float32),
        scratch_shapes=[
            pltpu.VMEM((N, DIM), jnp.float32),
            pltpu.VMEM((N, NB), jnp.float32),
        ],
    )(x, pos, norm1_w.reshape(1, DIM), norm2_w.reshape(1, DIM),
      w1_w.astype(jnp.bfloat16), w1_b.reshape(1, HID),
      w2_w.astype(jnp.bfloat16), w2_b.reshape(1, HID),
      w3_w.astype(jnp.bfloat16), w3_b.reshape(1, DIM))
    return out


# speedup vs baseline: 1.0067x; 1.0067x over previous
import jax
import jax.numpy as jnp
from jax.experimental import pallas as pl
from jax.experimental.pallas import tpu as pltpu

DIM = 128
BALL = 128
N = 2048
NB = N // BALL
HID = DIM * 4
EPS = float(jnp.finfo(jnp.float32).eps)
SCALE = DIM ** -0.5
NEG = -1e30

QT = 512
GRID = N // QT

_DN = (((1,), (1,)), ((), ()))


def _body(x_ref, pos_ref, n1_ref, n2_ref, w1w_ref, w1b_ref, w2w_ref,
          w2b_ref, w3w_ref, w3b_ref, o_ref, x1_ref, sel_ref):
    i = pl.program_id(0)

    @pl.when(i == 0)
    def _prep():
        x = x_ref[...]
        p = pos_ref[...].reshape(NB, BALL)
        rel = (p - jnp.mean(p, axis=1, keepdims=True)).reshape(N, 1)
        v = jnp.mean(x * x, axis=-1, keepdims=True)
        x1 = x * jax.lax.rsqrt(v + EPS) * n1_ref[...] + rel
        x1_ref[...] = x1
        bm = jnp.mean(x1.reshape(NB, BALL, DIM), axis=1)
        sim = jax.lax.dot_general(x1, bm, _DN,
                                  preferred_element_type=jnp.float32)
        idx = jax.lax.broadcasted_iota(jnp.int32, (N, NB), 1)
        m1 = jnp.max(sim, axis=-1, keepdims=True)
        i1 = jnp.min(jnp.where(sim == m1, idx, NB), axis=-1, keepdims=True)
        sel1 = idx == i1
        sim2 = jnp.where(sel1, NEG, sim)
        m2 = jnp.max(sim2, axis=-1, keepdims=True)
        i2 = jnp.min(jnp.where(sim2 == m2, idx, NB), axis=-1, keepdims=True)
        sel_ref[...] = jnp.where(sel1 | (idx == i2), 0.0, NEG)

    x1 = x1_ref[...]
    xq = x1_ref[pl.ds(i * QT, QT), :]

    rr = jax.lax.broadcasted_iota(jnp.int32, (NB, N), 0)
    cc = jax.lax.broadcasted_iota(jnp.int32, (NB, N), 1) // BALL
    colmap = (rr == cc).astype(jnp.bfloat16)
    bias = jax.lax.dot_general(
        sel_ref[pl.ds(i * QT, QT), :].astype(jnp.bfloat16), colmap,
        (((1,), (0,)), ((), ())),
        preferred_element_type=jnp.float32)

    s = jax.lax.dot_general(xq * SCALE, x1, _DN,
                            preferred_element_type=jnp.float32)
    s = s + bias
    m = jnp.max(s, axis=-1, keepdims=True)
    p = jnp.exp(s - m)
    attn = jax.lax.dot_general(p, x1, (((1,), (0,)), ((), ())),
                               preferred_element_type=jnp.float32)
    attn = attn / jnp.sum(p, axis=-1, keepdims=True)

    x2 = x_ref[pl.ds(i * QT, QT), :] + attn
    v2 = jnp.mean(x2 * x2, axis=-1, keepdims=True)
    xn = (x2 * jax.lax.rsqrt(v2 + EPS) * n2_ref[...]).astype(jnp.bfloat16)
    a = jax.lax.dot_general(xn, w1w_ref[...], _DN,
                            preferred_element_type=jnp.float32) + w1b_ref[...]
    b = jax.lax.dot_general(xn, w2w_ref[...], _DN,
                            preferred_element_type=jnp.float32) + w2b_ref[...]
    h = (b * (a * jax.nn.sigmoid(a))).astype(jnp.bfloat16)
    o_ref[...] = x2 + jax.lax.dot_general(
        h, w3w_ref[...], _DN, preferred_element_type=jnp.float32) + w3b_ref[...]


def kernel(x, pos, batch_idx, norm1_w, norm2_w, w1_w, w1_b, w2_w, w2_b,
           w3_w, w3_b):
    del batch_idx
    full = lambda shape: pl.BlockSpec(shape, lambda i: (0, 0))
    out = pl.pallas_call(
        _body,
        grid=(GRID,),
        in_specs=[
            full((N, DIM)),
            full((N, 1)),
            full((1, DIM)),
            full((1, DIM)),
            full((HID, DIM)),
            full((1, HID)),
            full((HID, DIM)),
            full((1, HID)),
            full((DIM, HID)),
            full((1, DIM)),
        ],
        out_specs=pl.BlockSpec((QT, DIM), lambda i: (i, 0)),
        out_shape=jax.ShapeDtypeStruct((N, DIM), jnp.float32),
        scratch_shapes=[
            pltpu.VMEM((N, DIM), jnp.float32),
            pltpu.VMEM((N, NB), jnp.float32),
        ],
    )(x, pos, norm1_w.reshape(1, DIM), norm2_w.reshape(1, DIM),
      w1_w.astype(jnp.bfloat16), w1_b.reshape(1, HID),
      w2_w.astype(jnp.bfloat16), w2_b.reshape(1, HID),
      w3_w.astype(jnp.bfloat16), w3_b.reshape(1, DIM))
    return out


# speedup vs baseline: 1.1746x; 1.1669x over previous
import jax
import jax.numpy as jnp
from jax.experimental import pallas as pl
from jax.experimental.pallas import tpu as pltpu

DIM = 128
BALL = 128
N = 2048
NB = N // BALL
HID = DIM * 4
EPS = float(jnp.finfo(jnp.float32).eps)
SCALE = DIM ** -0.5
NEG = -1e30

QT = 512
GRID = N // QT

_DN = (((1,), (1,)), ((), ()))


def _body(x_ref, pos_ref, n1_ref, n2_ref, w1w_ref, w1b_ref, w2w_ref,
          w2b_ref, w3w_ref, w3b_ref, o_ref, x1_ref, sel_ref, cmap_ref):
    i = pl.program_id(0)

    @pl.when(i == 0)
    def _prep():
        x = x_ref[...]
        p = pos_ref[...].reshape(NB, BALL)
        rel = (p - jnp.mean(p, axis=1, keepdims=True)).reshape(N, 1)
        v = jnp.mean(x * x, axis=-1, keepdims=True)
        x1 = x * jax.lax.rsqrt(v + EPS) * n1_ref[...] + rel
        x1_ref[...] = x1
        bm = jnp.mean(x1.reshape(NB, BALL, DIM), axis=1)
        simT = jax.lax.dot_general(bm, x1, _DN,
                                   preferred_element_type=jnp.float32)
        idx = jax.lax.broadcasted_iota(jnp.int32, (NB, N), 0)
        m1 = jnp.max(simT, axis=0, keepdims=True)
        i1 = jnp.min(jnp.where(simT == m1, idx, NB), axis=0, keepdims=True)
        sel1 = idx == i1
        sim2 = jnp.where(sel1, NEG, simT)
        m2 = jnp.max(sim2, axis=0, keepdims=True)
        i2 = jnp.min(jnp.where(sim2 == m2, idx, NB), axis=0, keepdims=True)
        sel_ref[...] = jnp.where(sel1 | (idx == i2), 0.0,
                                 NEG).astype(jnp.bfloat16)
        cc = jax.lax.broadcasted_iota(jnp.int32, (NB, N), 1) // BALL
        cmap_ref[...] = (idx == cc).astype(jnp.bfloat16)

    x1 = x1_ref[...]
    xq = x1_ref[pl.ds(i * QT, QT), :]

    bias = jax.lax.dot_general(
        sel_ref[:, pl.ds(i * QT, QT)], cmap_ref[...],
        (((0,), (0,)), ((), ())),
        preferred_element_type=jnp.float32)

    s = jax.lax.dot_general(xq * SCALE, x1, _DN,
                            preferred_element_type=jnp.float32)
    s = s + bias
    m = jnp.max(s, axis=-1, keepdims=True)
    p = jnp.exp(s - m)
    attn = jax.lax.dot_general(p, x1, (((1,), (0,)), ((), ())),
                               preferred_element_type=jnp.float32)
    attn = attn / jnp.sum(p, axis=-1, keepdims=True)

    x2 = x_ref[pl.ds(i * QT, QT), :] + attn
    v2 = jnp.mean(x2 * x2, axis=-1, keepdims=True)
    xn = x2 * jax.lax.rsqrt(v2 + EPS) * n2_ref[...]
    a = jax.lax.dot_general(xn, w1w_ref[...], _DN,
                            preferred_element_type=jnp.float32) + w1b_ref[...]
    b = jax.lax.dot_general(xn, w2w_ref[...], _DN,
                            preferred_element_type=jnp.float32) + w2b_ref[...]
    h = b * (a * jax.nn.sigmoid(a))
    o_ref[...] = x2 + jax.lax.dot_general(
        h, w3w_ref[...], _DN, preferred_element_type=jnp.float32) + w3b_ref[...]


def kernel(x, pos, batch_idx, norm1_w, norm2_w, w1_w, w1_b, w2_w, w2_b,
           w3_w, w3_b):
    del batch_idx
    full = lambda shape: pl.BlockSpec(shape, lambda i: (0, 0))
    out = pl.pallas_call(
        _body,
        grid=(GRID,),
        in_specs=[
            full((N, DIM)),
            full((N, 1)),
            full((1, DIM)),
            full((1, DIM)),
            full((HID, DIM)),
            full((1, HID)),
            full((HID, DIM)),
            full((1, HID)),
            full((DIM, HID)),
            full((1, DIM)),
        ],
        out_specs=pl.BlockSpec((QT, DIM), lambda i: (i, 0)),
        out_shape=jax.ShapeDtypeStruct((N, DIM), jnp.float32),
        scratch_shapes=[
            pltpu.VMEM((N, DIM), jnp.float32),
            pltpu.VMEM((NB, N), jnp.bfloat16),
            pltpu.VMEM((NB, N), jnp.bfloat16),
        ],
    )(x, pos, norm1_w.reshape(1, DIM), norm2_w.reshape(1, DIM),
      w1_w, w1_b.reshape(1, HID),
      w2_w, w2_b.reshape(1, HID),
      w3_w, w3_b.reshape(1, DIM))
    return out


# speedup vs baseline: 1.3336x; 1.1354x over previous
import jax
import jax.numpy as jnp
from jax.experimental import pallas as pl
from jax.experimental.pallas import tpu as pltpu

DIM = 128
BALL = 128
N = 2048
NB = N // BALL
HID = DIM * 4
EPS = float(jnp.finfo(jnp.float32).eps)
SCALE = DIM ** -0.5
NEG = -1e30

QT = 512
GRID = N // QT

_DN = (((1,), (1,)), ((), ()))


def _body(x_ref, pos_ref, n1_ref, n2_ref, w1w_ref, w1b_ref, w2w_ref,
          w2b_ref, w3w_ref, w3b_ref, o_ref, x1_ref, sel_ref, cmap_ref):
    i = pl.program_id(0)

    @pl.when(i == 0)
    def _prep():
        x = x_ref[...]
        p = pos_ref[...].reshape(NB, BALL)
        rel = (p - jnp.mean(p, axis=1, keepdims=True)).reshape(N, 1)
        v = jnp.mean(x * x, axis=-1, keepdims=True)
        x1 = x * jax.lax.rsqrt(v + EPS) * n1_ref[...] + rel
        x1_ref[...] = x1
        bm = jnp.mean(x1.reshape(NB, BALL, DIM), axis=1)
        simT = jax.lax.dot_general(bm, x1, _DN,
                                   preferred_element_type=jnp.float32)
        idx = jax.lax.broadcasted_iota(jnp.int32, (NB, N), 0)
        m1 = jnp.max(simT, axis=0, keepdims=True)
        i1 = jnp.min(jnp.where(simT == m1, idx, NB), axis=0, keepdims=True)
        sel1 = idx == i1
        sim2 = jnp.where(sel1, NEG, simT)
        m2 = jnp.max(sim2, axis=0, keepdims=True)
        i2 = jnp.min(jnp.where(sim2 == m2, idx, NB), axis=0, keepdims=True)
        sel_ref[...] = jnp.where(sel1 | (idx == i2), 1.0,
                                 0.0).astype(jnp.bfloat16)
        cc = jax.lax.broadcasted_iota(jnp.int32, (NB, N), 1) // BALL
        cmap_ref[...] = (idx == cc).astype(jnp.bfloat16)

    x1 = x1_ref[...]
    xq = x1_ref[pl.ds(i * QT, QT), :]

    mask = jax.lax.dot_general(
        sel_ref[:, pl.ds(i * QT, QT)], cmap_ref[...],
        (((0,), (0,)), ((), ())),
        preferred_element_type=jnp.float32)

    s = jax.lax.dot_general(xq * SCALE, x1, _DN,
                            preferred_element_type=jnp.float32)
    p = jnp.exp(s) * mask
    attn = jax.lax.dot_general(p, x1, (((1,), (0,)), ((), ())),
                               preferred_element_type=jnp.float32)
    attn = attn / jnp.sum(p, axis=-1, keepdims=True)

    x2 = x_ref[pl.ds(i * QT, QT), :] + attn
    v2 = jnp.mean(x2 * x2, axis=-1, keepdims=True)
    xn = x2 * jax.lax.rsqrt(v2 + EPS) * n2_ref[...]
    a = jax.lax.dot_general(xn, w1w_ref[...], _DN,
                            preferred_element_type=jnp.float32) + w1b_ref[...]
    b = jax.lax.dot_general(xn, w2w_ref[...], _DN,
                            preferred_element_type=jnp.float32) + w2b_ref[...]
    h = b * (a * jax.nn.sigmoid(a))
    o_ref[...] = x2 + jax.lax.dot_general(
        h, w3w_ref[...], _DN, preferred_element_type=jnp.float32) + w3b_ref[...]


def kernel(x, pos, batch_idx, norm1_w, norm2_w, w1_w, w1_b, w2_w, w2_b,
           w3_w, w3_b):
    del batch_idx
    full = lambda shape: pl.BlockSpec(shape, lambda i: (0, 0))
    out = pl.pallas_call(
        _body,
        grid=(GRID,),
        in_specs=[
            full((N, DIM)),
            full((N, 1)),
            full((1, DIM)),
            full((1, DIM)),
            full((HID, DIM)),
            full((1, HID)),
            full((HID, DIM)),
            full((1, HID)),
            full((DIM, HID)),
            full((1, DIM)),
        ],
        out_specs=pl.BlockSpec((QT, DIM), lambda i: (i, 0)),
        out_shape=jax.ShapeDtypeStruct((N, DIM), jnp.float32),
        scratch_shapes=[
            pltpu.VMEM((N, DIM), jnp.float32),
            pltpu.VMEM((NB, N), jnp.bfloat16),
            pltpu.VMEM((NB, N), jnp.bfloat16),
        ],
    )(x, pos, norm1_w.reshape(1, DIM), norm2_w.reshape(1, DIM),
      w1_w, w1_b.reshape(1, HID),
      w2_w, w2_b.reshape(1, HID),
      w3_w, w3_b.reshape(1, DIM))
    return out


# speedup vs baseline: 1.3887x; 1.0413x over previous
import jax
import jax.numpy as jnp
from jax.experimental import pallas as pl
from jax.experimental.pallas import tpu as pltpu

DIM = 128
BALL = 128
N = 2048
NB = N // BALL
HID = DIM * 4
EPS = float(jnp.finfo(jnp.float32).eps)
SCALE = DIM ** -0.5
NEG = -1e30

QT = 1024
GRID = N // QT

_DN = (((1,), (1,)), ((), ()))


def _body(x_ref, pos_ref, n1_ref, n2_ref, w1w_ref, w1b_ref, w2w_ref,
          w2b_ref, w3w_ref, w3b_ref, o_ref, x1_ref, sel_ref, cmap_ref):
    i = pl.program_id(0)

    @pl.when(i == 0)
    def _prep():
        x = x_ref[...]
        p = pos_ref[...].reshape(NB, BALL)
        rel = (p - jnp.mean(p, axis=1, keepdims=True)).reshape(N, 1)
        v = jnp.mean(x * x, axis=-1, keepdims=True)
        x1 = x * jax.lax.rsqrt(v + EPS) * n1_ref[...] + rel
        x1_ref[...] = x1
        bm = jnp.mean(x1.reshape(NB, BALL, DIM), axis=1)
        simT = jax.lax.dot_general(bm, x1, _DN,
                                   preferred_element_type=jnp.float32)
        idx = jax.lax.broadcasted_iota(jnp.int32, (NB, N), 0)
        m1 = jnp.max(simT, axis=0, keepdims=True)
        i1 = jnp.min(jnp.where(simT == m1, idx, NB), axis=0, keepdims=True)
        sel1 = idx == i1
        sim2 = jnp.where(sel1, NEG, simT)
        m2 = jnp.max(sim2, axis=0, keepdims=True)
        i2 = jnp.min(jnp.where(sim2 == m2, idx, NB), axis=0, keepdims=True)
        sel_ref[...] = jnp.where(sel1 | (idx == i2), 1.0,
                                 0.0).astype(jnp.bfloat16)
        cc = jax.lax.broadcasted_iota(jnp.int32, (NB, N), 1) // BALL
        cmap_ref[...] = (idx == cc).astype(jnp.bfloat16)

    x1 = x1_ref[...]
    xq = x1_ref[pl.ds(i * QT, QT), :]

    mask = jax.lax.dot_general(
        sel_ref[:, pl.ds(i * QT, QT)], cmap_ref[...],
        (((0,), (0,)), ((), ())),
        preferred_element_type=jnp.float32)

    s = jax.lax.dot_general(xq * SCALE, x1, _DN,
                            preferred_element_type=jnp.float32)
    p = jnp.exp(s) * mask
    attn = jax.lax.dot_general(p, x1, (((1,), (0,)), ((), ())),
                               preferred_element_type=jnp.float32)
    attn = attn / jnp.sum(p, axis=-1, keepdims=True)

    x2 = x_ref[pl.ds(i * QT, QT), :] + attn
    v2 = jnp.mean(x2 * x2, axis=-1, keepdims=True)
    xn = x2 * jax.lax.rsqrt(v2 + EPS) * n2_ref[...]
    a = jax.lax.dot_general(xn, w1w_ref[...], _DN,
                            preferred_element_type=jnp.float32) + w1b_ref[...]
    b = jax.lax.dot_general(xn, w2w_ref[...], _DN,
                            preferred_element_type=jnp.float32) + w2b_ref[...]
    h = b * (a * jax.nn.sigmoid(a))
    o_ref[...] = x2 + jax.lax.dot_general(
        h, w3w_ref[...], _DN, preferred_element_type=jnp.float32) + w3b_ref[...]


def kernel(x, pos, batch_idx, norm1_w, norm2_w, w1_w, w1_b, w2_w, w2_b,
           w3_w, w3_b):
    del batch_idx
    full = lambda shape: pl.BlockSpec(shape, lambda i: (0, 0))
    out = pl.pallas_call(
        _body,
        grid=(GRID,),
        in_specs=[
            full((N, DIM)),
            full((N, 1)),
            full((1, DIM)),
            full((1, DIM)),
            full((HID, DIM)),
            full((1, HID)),
            full((HID, DIM)),
            full((1, HID)),
            full((DIM, HID)),
            full((1, DIM)),
        ],
        out_specs=pl.BlockSpec((QT, DIM), lambda i: (i, 0)),
        out_shape=jax.ShapeDtypeStruct((N, DIM), jnp.float32),
        scratch_shapes=[
            pltpu.VMEM((N, DIM), jnp.float32),
            pltpu.VMEM((NB, N), jnp.bfloat16),
            pltpu.VMEM((NB, N), jnp.bfloat16),
        ],
    )(x, pos, norm1_w.reshape(1, DIM), norm2_w.reshape(1, DIM),
      w1_w, w1_b.reshape(1, HID),
      w2_w, w2_b.reshape(1, HID),
      w3_w, w3_b.reshape(1, DIM))
    return out


# speedup vs baseline: 1.4142x; 1.0184x over previous
import jax
import jax.numpy as jnp
from jax.experimental import pallas as pl
from jax.experimental.pallas import tpu as pltpu

DIM = 128
BALL = 128
N = 2048
NB = N // BALL
HID = DIM * 4
EPS = float(jnp.finfo(jnp.float32).eps)
SCALE = DIM ** -0.5
NEG = -1e30

QT = 2048
GRID = N // QT

_DN = (((1,), (1,)), ((), ()))


def _body(x_ref, pos_ref, n1_ref, n2_ref, w1w_ref, w1b_ref, w2w_ref,
          w2b_ref, w3w_ref, w3b_ref, o_ref, x1_ref, sel_ref, cmap_ref):
    i = pl.program_id(0)

    @pl.when(i == 0)
    def _prep():
        x = x_ref[...]
        p = pos_ref[...].reshape(NB, BALL)
        rel = (p - jnp.mean(p, axis=1, keepdims=True)).reshape(N, 1)
        v = jnp.mean(x * x, axis=-1, keepdims=True)
        x1 = x * jax.lax.rsqrt(v + EPS) * n1_ref[...] + rel
        x1_ref[...] = x1
        bm = jnp.mean(x1.reshape(NB, BALL, DIM), axis=1)
        simT = jax.lax.dot_general(bm, x1, _DN,
                                   preferred_element_type=jnp.float32)
        idx = jax.lax.broadcasted_iota(jnp.int32, (NB, N), 0)
        m1 = jnp.max(simT, axis=0, keepdims=True)
        i1 = jnp.min(jnp.where(simT == m1, idx, NB), axis=0, keepdims=True)
        sel1 = idx == i1
        sim2 = jnp.where(sel1, NEG, simT)
        m2 = jnp.max(sim2, axis=0, keepdims=True)
        i2 = jnp.min(jnp.where(sim2 == m2, idx, NB), axis=0, keepdims=True)
        sel_ref[...] = jnp.where(sel1 | (idx == i2), 1.0,
                                 0.0).astype(jnp.bfloat16)
        cc = jax.lax.broadcasted_iota(jnp.int32, (NB, N), 1) // BALL
        cmap_ref[...] = (idx == cc).astype(jnp.bfloat16)

    x1 = x1_ref[...]
    xq = x1_ref[pl.ds(i * QT, QT), :]

    mask = jax.lax.dot_general(
        sel_ref[:, pl.ds(i * QT, QT)], cmap_ref[...],
        (((0,), (0,)), ((), ())),
        preferred_element_type=jnp.float32)

    s = jax.lax.dot_general(xq * SCALE, x1, _DN,
                            preferred_element_type=jnp.float32)
    p = jnp.exp(s) * mask
    attn = jax.lax.dot_general(p, x1, (((1,), (0,)), ((), ())),
                               preferred_element_type=jnp.float32)
    attn = attn / jnp.sum(p, axis=-1, keepdims=True)

    x2 = x_ref[pl.ds(i * QT, QT), :] + attn
    v2 = jnp.mean(x2 * x2, axis=-1, keepdims=True)
    xn = x2 * jax.lax.rsqrt(v2 + EPS) * n2_ref[...]
    a = jax.lax.dot_general(xn, w1w_ref[...], _DN,
                            preferred_element_type=jnp.float32) + w1b_ref[...]
    b = jax.lax.dot_general(xn, w2w_ref[...], _DN,
                            preferred_element_type=jnp.float32) + w2b_ref[...]
    h = b * (a * jax.nn.sigmoid(a))
    o_ref[...] = x2 + jax.lax.dot_general(
        h, w3w_ref[...], _DN, preferred_element_type=jnp.float32) + w3b_ref[...]


def kernel(x, pos, batch_idx, norm1_w, norm2_w, w1_w, w1_b, w2_w, w2_b,
           w3_w, w3_b):
    del batch_idx
    full = lambda shape: pl.BlockSpec(shape, lambda i: (0, 0))
    out = pl.pallas_call(
        _body,
        grid=(GRID,),
        in_specs=[
            full((N, DIM)),
            full((N, 1)),
            full((1, DIM)),
            full((1, DIM)),
            full((HID, DIM)),
            full((1, HID)),
            full((HID, DIM)),
            full((1, HID)),
            full((DIM, HID)),
            full((1, DIM)),
        ],
        out_specs=pl.BlockSpec((QT, DIM), lambda i: (i, 0)),
        out_shape=jax.ShapeDtypeStruct((N, DIM), jnp.float32),
        scratch_shapes=[
            pltpu.VMEM((N, DIM), jnp.float32),
            pltpu.VMEM((NB, N), jnp.bfloat16),
            pltpu.VMEM((NB, N), jnp.bfloat16),
        ],
    )(x, pos, norm1_w.reshape(1, DIM), norm2_w.reshape(1, DIM),
      w1_w, w1_b.reshape(1, HID),
      w2_w, w2_b.reshape(1, HID),
      w3_w, w3_b.reshape(1, DIM))
    return out
